# baseline (speedup 1.0000x reference)
import jax
import jax.numpy as jnp
from jax.experimental import pallas as pl
from jax.experimental.pallas import tpu as pltpu

NMS_SIZE = 5
MIN_VAL = 1e-05
H = 512
W = 512


def _max5_rows(x):
    z1 = jnp.zeros((1, W), dtype=x.dtype)
    t = jnp.maximum(x, jnp.concatenate([x[1:], z1], axis=0))
    return jnp.maximum(x, jnp.maximum(
        jnp.concatenate([z1, t[:1], t[:-2]], axis=0),
        jnp.concatenate([t[1:], z1], axis=0)))


def _max5_cols(x):
    z1 = jnp.zeros((H, 1), dtype=x.dtype)
    t = jnp.maximum(x, jnp.concatenate([x[:, 1:], z1], axis=1))
    return jnp.maximum(x, jnp.maximum(
        jnp.concatenate([z1, t[:, :1], t[:, :-2]], axis=1),
        jnp.concatenate([t[:, 1:], z1], axis=1)))


def _nms_body(scale_ref, center_ref, x_hbm, wc_hbm, mask_hbm,
              xbuf, wcbuf, mbuf, insem, wcsem, msem):
    B = x_hbm.shape[0]
    col = jax.lax.broadcasted_iota(jnp.int32, (H, W), 1).astype(jnp.float32)
    row = jax.lax.broadcasted_iota(jnp.int32, (H, W), 0).astype(jnp.float32)

    def in_copy(b):
        return pltpu.make_async_copy(x_hbm.at[b, 0], xbuf.at[b], insem.at[b])

    def out_copies(b):
        return (pltpu.make_async_copy(wcbuf.at[b], wc_hbm.at[b], wcsem.at[b]),
                pltpu.make_async_copy(mbuf.at[b], mask_hbm.at[b], msem.at[b]))

    for b in range(B):
        in_copy(b).start()
    for b in range(B):
        in_copy(b).wait()

        x = xbuf[b]
        m = _max5_cols(_max5_rows(x))
        mask = (x > MIN_VAL) & (x >= m)
        s = scale_ref[b]
        cx2 = center_ref[2 * b] - (W / 2.0) * s
        cy2 = center_ref[2 * b + 1] + (H / 2.0) * s
        wcbuf[b, 0] = jnp.where(mask, col * s + cx2, 0.0)
        wcbuf[b, 1] = jnp.where(mask, row * (-s) + cy2, 0.0)
        mbuf[b] = mask.astype(jnp.int8)

        cwc, cm = out_copies(b)
        cwc.start()
        cm.start()

    for b in range(B):
        cwc, cm = out_copies(b)
        cwc.wait()
        cm.wait()


def kernel(input_map, bev_scale, bev_center):
    B = input_map.shape[0]
    wc, mask = pl.pallas_call(
        _nms_body,
        in_specs=[
            pl.BlockSpec(memory_space=pltpu.SMEM),
            pl.BlockSpec(memory_space=pltpu.SMEM),
            pl.BlockSpec(memory_space=pltpu.MemorySpace.HBM),
        ],
        out_specs=[
            pl.BlockSpec(memory_space=pltpu.MemorySpace.HBM),
            pl.BlockSpec(memory_space=pltpu.MemorySpace.HBM),
        ],
        out_shape=[
            jax.ShapeDtypeStruct((B, 2, H, W), jnp.float32),
            jax.ShapeDtypeStruct((B, H, W), jnp.int8),
        ],
        scratch_shapes=[
            pltpu.VMEM((4, H, W), jnp.float32),
            pltpu.VMEM((4, 2, H, W), jnp.float32),
            pltpu.VMEM((4, H, W), jnp.int8),
            pltpu.SemaphoreType.DMA((4,)),
            pltpu.SemaphoreType.DMA((4,)),
            pltpu.SemaphoreType.DMA((4,)),
        ],
    )(bev_scale, bev_center.reshape(-1), input_map)
    return wc, mask.view(jnp.bool_)
